# baseline (speedup 1.0000x reference)
import functools

import jax
import jax.numpy as jnp
from jax import lax
from jax.experimental import pallas as pl
from jax.experimental.pallas import tpu as pltpu
from jax.experimental.pallas import tpu_sc as plsc

_POS_OV = 0.7
_NEG_OV = 0.3
_NUM_FG = 256
_NUM_BG = 256
_INT_MIN = -2147483648
_NEG_INF_KEY = -2139095041


def _assign_kernel(a_ref, score_ref, gt_ref, gl_ref, clst_ref, reg_ref,
                   clsw_ref, regw_ref, skey_ref):
  B, M, _ = gt_ref.shape
  K = a_ref.shape[1]
  f32 = jnp.float32

  ax1 = a_ref[0:1, :]
  ay1 = a_ref[1:2, :]
  ax2 = a_ref[2:3, :]
  ay2 = a_ref[3:4, :]
  area_a = jnp.maximum(ax2 - ax1, 0.0) * jnp.maximum(ay2 - ay1, 0.0)
  aw = jnp.maximum(ax2 - ax1, 1e-6)
  ah = jnp.maximum(ay2 - ay1, 1e-6)
  axc = ax1 + 0.5 * aw
  ayc = ay1 + 0.5 * ah
  midx = lax.broadcasted_iota(jnp.int32, (M, K), 0)

  cls_t_rows = []
  cls_w_rows = []
  reg_w_rows = []
  for b in range(B):
    g = gt_ref[b]
    gl = gl_ref[b]
    gx1 = g[:, 0:1]
    gy1 = g[:, 1:2]
    gx2 = g[:, 2:3]
    gy2 = g[:, 3:4]
    x1 = jnp.maximum(ax1, gx1)
    y1 = jnp.maximum(ay1, gy1)
    x2 = jnp.minimum(ax2, gx2)
    y2 = jnp.minimum(ay2, gy2)
    inter = jnp.maximum(x2 - x1, 0.0) * jnp.maximum(y2 - y1, 0.0)
    area_g = jnp.maximum(gx2 - gx1, 0.0) * jnp.maximum(gy2 - gy1, 0.0)
    union = area_a + area_g - inter
    ov = inter / jnp.maximum(union, 1e-8)

    max_ov = jnp.max(ov, axis=0, keepdims=True)
    arg = jnp.min(jnp.where(ov == max_ov, midx, M), axis=0, keepdims=True)
    gt_max = jnp.max(ov, axis=1, keepdims=True)
    near_best = jnp.where(ov >= gt_max - 1e-5, 1.0, 0.0)
    is_best = (jnp.max(near_best, axis=0, keepdims=True) > 0.0) & (max_ov > 0.0)

    onehot = (midx == arg).astype(f32)
    gcols = jnp.concatenate([gl, gx1, gy1, gx2, gy2], axis=1)
    gath = jax.lax.dot_general(
        gcols, onehot, (((0,), (0,)), ((), ())),
        precision=lax.Precision.HIGHEST,
        preferred_element_type=f32)
    glab = gath[0:1, :]
    gx1g = gath[1:2, :]
    gy1g = gath[2:3, :]
    gx2g = gath[3:4, :]
    gy2g = gath[4:5, :]

    labels = jnp.where(max_ov < _NEG_OV, 0.0, -1.0)
    labels = jnp.where(is_best, 1.0, labels)
    labels = jnp.where(max_ov >= _POS_OV, 1.0, labels)
    cls_t = jnp.where(labels == 1.0, glab, labels)

    gw = jnp.maximum(gx2g - gx1g, 1e-6)
    gh = jnp.maximum(gy2g - gy1g, 1e-6)
    gxc = gx1g + 0.5 * gw
    gyc = gy1g + 0.5 * gh
    tx = (gxc - axc) / aw
    ty = (gyc - ayc) / ah
    tw = jnp.log(gw / aw)
    th = jnp.log(gh / ah)
    reg_ref[b] = jnp.concatenate([tx, ty, tw, th], axis=0)

    cls_w = (labels >= 0.0).astype(f32)
    reg_w = (labels == 1.0).astype(f32)
    clst_ref[b:b + 1, :] = cls_t
    cls_t_rows.append(cls_t)
    cls_w_rows.append(cls_w)
    reg_w_rows.append(reg_w)

  cls_t_all = jnp.concatenate(cls_t_rows, axis=0)
  cls_w_all = jnp.concatenate(cls_w_rows, axis=0)
  reg_w_all = jnp.concatenate(reg_w_rows, axis=0)
  score = score_ref[...]

  fg_elig = ((cls_t_all > 0.0) & (cls_w_all > 0.0)).astype(f32)
  bg_elig = ((cls_t_all == 0.0) & (cls_w_all > 0.0)).astype(f32)
  elig = jnp.concatenate([fg_elig, bg_elig], axis=0) > 0.0
  s_f = jnp.where(elig, jnp.concatenate([score, score], axis=0),
                  -jnp.inf)
  bits = lax.bitcast_convert_type(s_f, jnp.int32)
  skey_ref[...] = jnp.where(bits >= 0, bits, bits ^ jnp.int32(0x7FFFFFFF))
  clsw_ref[...] = cls_w_all
  regw_ref[...] = reg_w_all


def _lane_sum(x):
  total = x[0]
  for l in range(1, 16):
    total = total + x[l]
  return total


def _count_ge(keys_v, nchunk, cand):
  def body(i, accs):
    base = i * 80
    new = []
    for j, a in enumerate(accs):
      v = keys_v[pl.ds(base + j * 16, 16)]
      new.append(a + jnp.where(v >= cand, 1, 0).astype(jnp.int32))
    return tuple(new)

  zero = jnp.zeros((16,), jnp.int32)
  accs = lax.fori_loop(0, nchunk // 5, body, (zero,) * 5)
  total = accs[0] + accs[1] + accs[2] + accs[3] + accs[4]
  return _lane_sum(total)


def _lane_prefix_excl(x):
  idx = lax.broadcasted_iota(jnp.int32, (16,), 0)
  dnums = lax.GatherDimensionNumbers(
      offset_dims=(), collapsed_slice_dims=(0,), start_index_map=(0,))
  incl = x
  for step in (1, 2, 4, 8):
    j = jnp.maximum(idx - step, 0)
    shifted = lax.gather(incl, j[:, None], dnums, (1,),
                         mode=lax.GatherScatterMode.PROMISE_IN_BOUNDS)
    incl = incl + jnp.where(idx >= step, shifted, 0)
  return incl - x


def _subsample_sc_body(skey_hbm, clsw_hbm, regw_hbm, clsw_out, regw_out,
                       keys_v, mask_v, tmp_v, shared, *, num_fg, num_bg, K):
  cid = lax.axis_index("c")
  sid = lax.axis_index("s")
  nchunk = K // 16
  active = (cid == 0) & (sid < 8)

  @pl.when(active)
  def _phase1():
    kk = jnp.where(sid < 4, num_fg, num_bg)
    pltpu.sync_copy(skey_hbm.at[sid], keys_v)

    def bit_body(_, carry):
      prefix, bit = carry
      cand = prefix | bit
      cnt = _count_ge(keys_v, nchunk, cand ^ jnp.int32(_INT_MIN))
      prefix = jnp.where(cnt >= kk, cand, prefix)
      return prefix, lax.shift_right_logical(bit, 1)

    prefix, _ = lax.fori_loop(
        0, 32, bit_body, (jnp.int32(0), jnp.int32(_INT_MIN)))
    thr = prefix ^ jnp.int32(_INT_MIN)
    n_gt = _count_ge(keys_v, nchunk, thr + 1)
    need = kk - n_gt

    def mask_body(i, tcnt):
      v = keys_v[pl.ds(i * 16, 16)]
      tie = v == thr
      ntie = plsc.all_reduce_population_count(tie)[0]
      limit = need - tcnt
      boundary = (tcnt < need) & ((tcnt + ntie) > need)
      excl = lax.cond(
          boundary,
          lambda: _lane_prefix_excl(tie.astype(jnp.int32)),
          lambda: jnp.zeros((16,), jnp.int32))
      sel = (v > thr) | (tie & (excl < limit))
      sel = sel & (v != _NEG_INF_KEY)
      mask_v[pl.ds(i * 16, 16)] = sel.astype(jnp.float32)
      return tcnt + ntie

    lax.fori_loop(0, nchunk, mask_body, jnp.int32(0))
    pltpu.sync_copy(mask_v, shared.at[sid])

  plsc.subcore_barrier()

  @pl.when(active & (sid < 4))
  def _phase2():
    pltpu.sync_copy(shared.at[sid + 4], tmp_v)

    def or_body(i, _):
      d = pl.ds(i * 16, 16)
      mask_v[d] = jnp.maximum(mask_v[d], tmp_v[d])
      return 0

    lax.fori_loop(0, nchunk, or_body, 0)

    for w_hbm, w_out in ((clsw_hbm, clsw_out), (regw_hbm, regw_out)):
      pltpu.sync_copy(w_hbm.at[sid], tmp_v)

      def mul_body(i, _):
        d = pl.ds(i * 16, 16)
        tmp_v[d] = tmp_v[d] * mask_v[d]
        return 0

      lax.fori_loop(0, nchunk, mul_body, 0)
      pltpu.sync_copy(tmp_v, w_out.at[sid])


def _subsample_sc(skey, cls_w, reg_w):
  R, K = skey.shape
  B = R // 2
  mesh = plsc.VectorSubcoreMesh(core_axis_name="c", subcore_axis_name="s")
  body = functools.partial(_subsample_sc_body, num_fg=_NUM_FG, num_bg=_NUM_BG,
                           K=K)
  f = pl.kernel(
      body,
      mesh=mesh,
      compiler_params=pltpu.CompilerParams(needs_layout_passes=False),
      out_type=(
          jax.ShapeDtypeStruct((B, K), jnp.float32),
          jax.ShapeDtypeStruct((B, K), jnp.float32),
      ),
      scratch_types=[
          pltpu.VMEM((K,), jnp.int32),
          pltpu.VMEM((K,), jnp.float32),
          pltpu.VMEM((K,), jnp.float32),
          pltpu.VMEM_SHARED((8, K), jnp.float32),
      ],
  )
  return f(skey, cls_w, reg_w)


@jax.jit
def kernel(anchors, rpn_cls_score, gt_boxes, gt_labels):
  K = anchors.shape[0]
  B, M, _ = gt_boxes.shape
  anchors_t = anchors.T
  gl_f = gt_labels.astype(jnp.float32)[..., None]
  cls_t, reg, cls_w_raw, reg_w_raw, skey = pl.pallas_call(
      _assign_kernel,
      out_shape=(
          jax.ShapeDtypeStruct((B, K), jnp.float32),
          jax.ShapeDtypeStruct((B, 4, K), jnp.float32),
          jax.ShapeDtypeStruct((B, K), jnp.float32),
          jax.ShapeDtypeStruct((B, K), jnp.float32),
          jax.ShapeDtypeStruct((2 * B, K), jnp.int32),
      ),
  )(anchors_t, rpn_cls_score, gt_boxes, gl_f)
  cls_w, reg_w = _subsample_sc(skey, cls_w_raw, reg_w_raw)
  reg_t = jnp.transpose(reg, (0, 2, 1))
  return cls_t, reg_t, cls_w, reg_w


# speedup vs baseline: 1.0651x; 1.0651x over previous
import functools

import jax
import jax.numpy as jnp
from jax import lax
from jax.experimental import pallas as pl
from jax.experimental.pallas import tpu as pltpu
from jax.experimental.pallas import tpu_sc as plsc

_POS_OV = 0.7
_NEG_OV = 0.3
_NUM_FG = 256
_NUM_BG = 256
_INT_MIN = -2147483648
_NEG_INF_KEY = -2139095041


def _assign_kernel(a_ref, score_ref, gt_ref, gl_ref, clst_ref, reg_ref,
                   clsw_ref, regw_ref, skey_ref):
  B, M, _ = gt_ref.shape
  K = a_ref.shape[1]
  f32 = jnp.float32

  ax1 = a_ref[0:1, :]
  ay1 = a_ref[1:2, :]
  ax2 = a_ref[2:3, :]
  ay2 = a_ref[3:4, :]
  area_a = jnp.maximum(ax2 - ax1, 0.0) * jnp.maximum(ay2 - ay1, 0.0)
  aw = jnp.maximum(ax2 - ax1, 1e-6)
  ah = jnp.maximum(ay2 - ay1, 1e-6)
  axc = ax1 + 0.5 * aw
  ayc = ay1 + 0.5 * ah

  cls_t_rows = []
  cls_w_rows = []
  reg_w_rows = []
  for b in range(B):
    g = gt_ref[b]
    gl = gl_ref[b]
    gx1 = g[:, 0:1]
    gy1 = g[:, 1:2]
    gx2 = g[:, 2:3]
    gy2 = g[:, 3:4]
    x1 = jnp.maximum(ax1, gx1)
    y1 = jnp.maximum(ay1, gy1)
    x2 = jnp.minimum(ax2, gx2)
    y2 = jnp.minimum(ay2, gy2)
    inter = jnp.maximum(x2 - x1, 0.0) * jnp.maximum(y2 - y1, 0.0)
    area_g = jnp.maximum(gx2 - gx1, 0.0) * jnp.maximum(gy2 - gy1, 0.0)
    union = area_a + area_g - inter
    ov = inter / jnp.maximum(union, 1e-8)

    max_ov = jnp.max(ov, axis=0, keepdims=True)
    gt_max = jnp.max(ov, axis=1, keepdims=True)
    near_best = jnp.where(ov >= gt_max - 1e-5, 1.0, 0.0)
    is_best = (jnp.max(near_best, axis=0, keepdims=True) > 0.0) & (max_ov > 0.0)

    onehot_any = (ov == max_ov).astype(f32)
    tri = (jax.lax.broadcasted_iota(jnp.int32, (M, M), 1)
           < jax.lax.broadcasted_iota(jnp.int32, (M, M), 0)).astype(f32)
    precnt = jax.lax.dot_general(
        tri, onehot_any, (((1,), (0,)), ((), ())),
        preferred_element_type=f32)
    onehot = jnp.where(precnt > 0.0, 0.0, onehot_any)
    gcols = jnp.concatenate([gl, gx1, gy1, gx2, gy2], axis=1)
    gath = jax.lax.dot_general(
        gcols, onehot, (((0,), (0,)), ((), ())),
        precision=lax.Precision.HIGHEST,
        preferred_element_type=f32)
    glab = gath[0:1, :]
    gx1g = gath[1:2, :]
    gy1g = gath[2:3, :]
    gx2g = gath[3:4, :]
    gy2g = gath[4:5, :]

    labels = jnp.where(max_ov < _NEG_OV, 0.0, -1.0)
    labels = jnp.where(is_best, 1.0, labels)
    labels = jnp.where(max_ov >= _POS_OV, 1.0, labels)
    cls_t = jnp.where(labels == 1.0, glab, labels)

    gw = jnp.maximum(gx2g - gx1g, 1e-6)
    gh = jnp.maximum(gy2g - gy1g, 1e-6)
    gxc = gx1g + 0.5 * gw
    gyc = gy1g + 0.5 * gh
    tx = (gxc - axc) / aw
    ty = (gyc - ayc) / ah
    tw = jnp.log(gw / aw)
    th = jnp.log(gh / ah)
    reg_ref[b] = jnp.concatenate([tx, ty, tw, th], axis=0)

    cls_w = (labels >= 0.0).astype(f32)
    reg_w = (labels == 1.0).astype(f32)
    clst_ref[b:b + 1, :] = cls_t
    cls_t_rows.append(cls_t)
    cls_w_rows.append(cls_w)
    reg_w_rows.append(reg_w)

  cls_t_all = jnp.concatenate(cls_t_rows, axis=0)
  cls_w_all = jnp.concatenate(cls_w_rows, axis=0)
  reg_w_all = jnp.concatenate(reg_w_rows, axis=0)
  score = score_ref[...]

  fg_elig = ((cls_t_all > 0.0) & (cls_w_all > 0.0)).astype(f32)
  bg_elig = ((cls_t_all == 0.0) & (cls_w_all > 0.0)).astype(f32)
  elig = jnp.concatenate([fg_elig, bg_elig], axis=0) > 0.0
  s_f = jnp.where(elig, jnp.concatenate([score, score], axis=0),
                  -jnp.inf)
  bits = lax.bitcast_convert_type(s_f, jnp.int32)
  skey_ref[...] = jnp.where(bits >= 0, bits, bits ^ jnp.int32(0x7FFFFFFF))
  clsw_ref[...] = cls_w_all
  regw_ref[...] = reg_w_all


def _lane_sum(x):
  total = x[0]
  for l in range(1, 16):
    total = total + x[l]
  return total


def _count_ge(keys_v, nchunk, cand):
  def body(i, accs):
    base = i * 80
    new = []
    for j, a in enumerate(accs):
      v = keys_v[pl.ds(base + j * 16, 16)]
      new.append(a + jnp.where(v >= cand, 1, 0).astype(jnp.int32))
    return tuple(new)

  zero = jnp.zeros((16,), jnp.int32)
  accs = lax.fori_loop(0, nchunk // 5, body, (zero,) * 5)
  total = accs[0] + accs[1] + accs[2] + accs[3] + accs[4]
  return _lane_sum(total)


def _lane_prefix_excl(x):
  idx = lax.broadcasted_iota(jnp.int32, (16,), 0)
  dnums = lax.GatherDimensionNumbers(
      offset_dims=(), collapsed_slice_dims=(0,), start_index_map=(0,))
  incl = x
  for step in (1, 2, 4, 8):
    j = jnp.maximum(idx - step, 0)
    shifted = lax.gather(incl, j[:, None], dnums, (1,),
                         mode=lax.GatherScatterMode.PROMISE_IN_BOUNDS)
    incl = incl + jnp.where(idx >= step, shifted, 0)
  return incl - x


def _subsample_sc_body(skey_hbm, clsw_hbm, regw_hbm, clsw_out, regw_out,
                       keys_v, bkey_v, bpos_v, mask_v, tmp_v, shared, *,
                       num_fg, num_bg, K):
  cid = lax.axis_index("c")
  sid = lax.axis_index("s")
  nchunk = K // 16
  active = (cid == 0) & (sid < 8)
  IM = jnp.int32(_INT_MIN)

  def bit_const(p):
    return jnp.int32(_INT_MIN) if p == 31 else jnp.int32(1 << p)

  def count_ge_band(nchunks, cand_s):
    def body(i, acc):
      v = bkey_v[pl.ds(i * 16, 16)]
      return acc + jnp.where(v >= cand_s, 1, 0).astype(jnp.int32)

    acc = lax.fori_loop(0, nchunks, body, jnp.zeros((16,), jnp.int32))
    return _lane_sum(acc)

  def compact(nchunks, shift, prefix, from_full):
    pshift = lax.shift_right_logical(prefix, shift)
    iota16 = lax.broadcasted_iota(jnp.int32, (16,), 0)

    def body(i, carry):
      off, bminv, bmaxv = carry
      if from_full:
        v = keys_v[pl.ds(i * 16, 16)]
        p = iota16 + i * 16
      else:
        v = bkey_v[pl.ds(i * 16, 16)]
        p = bpos_v[pl.ds(i * 16, 16)]
      u = v ^ IM
      match = (lax.shift_right_logical(u, shift) == pshift) & (v != IM)
      plsc.store_compressed(bkey_v.at[pl.ds(off, 16)], v, mask=match)
      plsc.store_compressed(bpos_v.at[pl.ds(off, 16)], p, mask=match)
      npop = plsc.all_reduce_population_count(match)[0]
      bminv = jnp.minimum(bminv, jnp.where(match, v, 2147483647))
      bmaxv = jnp.maximum(bmaxv, jnp.where(match, v, _INT_MIN))
      return off + npop, bminv, bmaxv

    off, bminv, bmaxv = lax.fori_loop(
        0, nchunks,
        body,
        (jnp.int32(0), jnp.full((16,), 2147483647, jnp.int32),
         jnp.full((16,), _INT_MIN, jnp.int32)))
    bkey_v[pl.ds(off, 16)] = jnp.full((16,), _INT_MIN, jnp.int32)
    bmin = bminv[0]
    bmax = bmaxv[0]
    for l in range(1, 16):
      bmin = jnp.minimum(bmin, bminv[l])
      bmax = jnp.maximum(bmax, bmaxv[l])
    return off, bmin, bmax

  @pl.when(active)
  def _phase1():
    kk = jnp.where(sid < 4, num_fg, num_bg)
    pltpu.sync_copy(skey_hbm.at[sid], keys_v)

    prefix = jnp.int32(0)
    krem = kk
    n_above = jnp.int32(0)
    nband = jnp.int32(K)
    for shift in (28, 24, 20, 16, 12, 8, 4, 0):
      for p in range(shift + 3, shift - 1, -1):
        cand = prefix | bit_const(p)
        cand_s = cand ^ IM
        if shift == 28:
          cnt = _count_ge(keys_v, nchunk, cand_s)
        else:
          cnt = count_ge_band((nband + 15) >> 4, cand_s)
        band_cnt = cnt - n_above
        take = band_cnt >= krem
        prefix = jnp.where(take, cand, prefix)
        krem = jnp.where(take, krem, krem - band_cnt)
        n_above = jnp.where(take, n_above, cnt)
      if shift > 0:
        nin = nchunk if shift == 28 else (nband + 15) >> 4
        nband, bmin, bmax = compact(nin, shift, prefix, shift == 28)
        n_above = jnp.int32(0)
        collapse = (nband > 0) & (bmin == bmax)
        prefix = jnp.where(collapse, bmin ^ IM, prefix)
        nband = jnp.where(collapse, 0, nband)

    thr = prefix ^ IM
    need = krem

    def mask_body(i, tcnt):
      v = keys_v[pl.ds(i * 16, 16)]
      tie = v == thr
      ntie = plsc.all_reduce_population_count(tie)[0]
      limit = need - tcnt
      boundary = (tcnt < need) & ((tcnt + ntie) > need)
      excl = lax.cond(
          boundary,
          lambda: _lane_prefix_excl(tie.astype(jnp.int32)),
          lambda: jnp.zeros((16,), jnp.int32))
      sel = (v > thr) | (tie & (excl < limit))
      sel = sel & (v != _NEG_INF_KEY)
      mask_v[pl.ds(i * 16, 16)] = sel.astype(jnp.float32)
      return tcnt + ntie

    lax.fori_loop(0, nchunk, mask_body, jnp.int32(0))
    pltpu.sync_copy(mask_v, shared.at[sid])

  plsc.subcore_barrier()

  @pl.when(active & (sid < 4))
  def _phase2():
    pltpu.sync_copy(shared.at[sid + 4], tmp_v)

    def or_body(i, _):
      d = pl.ds(i * 16, 16)
      mask_v[d] = jnp.maximum(mask_v[d], tmp_v[d])
      return 0

    lax.fori_loop(0, nchunk, or_body, 0)

    for w_hbm, w_out in ((clsw_hbm, clsw_out), (regw_hbm, regw_out)):
      pltpu.sync_copy(w_hbm.at[sid], tmp_v)

      def mul_body(i, _):
        d = pl.ds(i * 16, 16)
        tmp_v[d] = tmp_v[d] * mask_v[d]
        return 0

      lax.fori_loop(0, nchunk, mul_body, 0)
      pltpu.sync_copy(tmp_v, w_out.at[sid])


def _subsample_sc(skey, cls_w, reg_w):
  R, K = skey.shape
  B = R // 2
  mesh = plsc.VectorSubcoreMesh(core_axis_name="c", subcore_axis_name="s")
  body = functools.partial(_subsample_sc_body, num_fg=_NUM_FG, num_bg=_NUM_BG,
                           K=K)
  f = pl.kernel(
      body,
      mesh=mesh,
      compiler_params=pltpu.CompilerParams(needs_layout_passes=False),
      out_type=(
          jax.ShapeDtypeStruct((B, K), jnp.float32),
          jax.ShapeDtypeStruct((B, K), jnp.float32),
      ),
      scratch_types=[
          pltpu.VMEM((K,), jnp.int32),
          pltpu.VMEM((K + 16,), jnp.int32),
          pltpu.VMEM((K + 16,), jnp.int32),
          pltpu.VMEM((K,), jnp.float32),
          pltpu.VMEM((K,), jnp.float32),
          pltpu.VMEM_SHARED((8, K), jnp.float32),
      ],
  )
  return f(skey, cls_w, reg_w)


@jax.jit
def kernel(anchors, rpn_cls_score, gt_boxes, gt_labels):
  K = anchors.shape[0]
  B, M, _ = gt_boxes.shape
  anchors_t = anchors.T
  gl_f = gt_labels.astype(jnp.float32)[..., None]
  cls_t, reg, cls_w_raw, reg_w_raw, skey = pl.pallas_call(
      _assign_kernel,
      out_shape=(
          jax.ShapeDtypeStruct((B, K), jnp.float32),
          jax.ShapeDtypeStruct((B, 4, K), jnp.float32),
          jax.ShapeDtypeStruct((B, K), jnp.float32),
          jax.ShapeDtypeStruct((B, K), jnp.float32),
          jax.ShapeDtypeStruct((2 * B, K), jnp.int32),
      ),
  )(anchors_t, rpn_cls_score, gt_boxes, gl_f)
  cls_w, reg_w = _subsample_sc(skey, cls_w_raw, reg_w_raw)
  reg_t = jnp.transpose(reg, (0, 2, 1))
  return cls_t, reg_t, cls_w, reg_w


# speedup vs baseline: 1.3032x; 1.2236x over previous
import functools

import jax
import jax.numpy as jnp
from jax import lax
from jax.experimental import pallas as pl
from jax.experimental.pallas import tpu as pltpu
from jax.experimental.pallas import tpu_sc as plsc

_POS_OV = 0.7
_NEG_OV = 0.3
_NUM_FG = 256
_NUM_BG = 256
_INT_MIN = -2147483648
_NEG_INF_KEY = -2139095041


def _assign_kernel(a_ref, score_ref, gt_ref, gl_ref, clst_ref, reg_ref,
                   clsw_ref, regw_ref, skey_ref):
  B, M, _ = gt_ref.shape
  K = a_ref.shape[1]
  f32 = jnp.float32

  ax1 = a_ref[0:1, :]
  ay1 = a_ref[1:2, :]
  ax2 = a_ref[2:3, :]
  ay2 = a_ref[3:4, :]
  area_a = jnp.maximum(ax2 - ax1, 0.0) * jnp.maximum(ay2 - ay1, 0.0)
  aw = jnp.maximum(ax2 - ax1, 1e-6)
  ah = jnp.maximum(ay2 - ay1, 1e-6)
  axc = ax1 + 0.5 * aw
  ayc = ay1 + 0.5 * ah

  cls_t_rows = []
  cls_w_rows = []
  reg_w_rows = []
  for b in range(B):
    g = gt_ref[b]
    gl = gl_ref[b]
    gx1 = g[:, 0:1]
    gy1 = g[:, 1:2]
    gx2 = g[:, 2:3]
    gy2 = g[:, 3:4]
    x1 = jnp.maximum(ax1, gx1)
    y1 = jnp.maximum(ay1, gy1)
    x2 = jnp.minimum(ax2, gx2)
    y2 = jnp.minimum(ay2, gy2)
    inter = jnp.maximum(x2 - x1, 0.0) * jnp.maximum(y2 - y1, 0.0)
    area_g = jnp.maximum(gx2 - gx1, 0.0) * jnp.maximum(gy2 - gy1, 0.0)
    union = area_a + area_g - inter
    ov = inter / jnp.maximum(union, 1e-8)

    max_ov = jnp.max(ov, axis=0, keepdims=True)
    gt_max = jnp.max(ov, axis=1, keepdims=True)
    near_best = jnp.where(ov >= gt_max - 1e-5, 1.0, 0.0)
    is_best = (jnp.max(near_best, axis=0, keepdims=True) > 0.0) & (max_ov > 0.0)

    onehot_any = (ov == max_ov).astype(f32)
    tri = (jax.lax.broadcasted_iota(jnp.int32, (M, M), 1)
           < jax.lax.broadcasted_iota(jnp.int32, (M, M), 0)).astype(f32)
    precnt = jax.lax.dot_general(
        tri, onehot_any, (((1,), (0,)), ((), ())),
        preferred_element_type=f32)
    onehot = jnp.where(precnt > 0.0, 0.0, onehot_any)
    gcols = jnp.concatenate([gl, gx1, gy1, gx2, gy2], axis=1)
    gath = jax.lax.dot_general(
        gcols, onehot, (((0,), (0,)), ((), ())),
        precision=lax.Precision.HIGHEST,
        preferred_element_type=f32)
    glab = gath[0:1, :]
    gx1g = gath[1:2, :]
    gy1g = gath[2:3, :]
    gx2g = gath[3:4, :]
    gy2g = gath[4:5, :]

    labels = jnp.where(max_ov < _NEG_OV, 0.0, -1.0)
    labels = jnp.where(is_best, 1.0, labels)
    labels = jnp.where(max_ov >= _POS_OV, 1.0, labels)
    cls_t = jnp.where(labels == 1.0, glab, labels)

    gw = jnp.maximum(gx2g - gx1g, 1e-6)
    gh = jnp.maximum(gy2g - gy1g, 1e-6)
    gxc = gx1g + 0.5 * gw
    gyc = gy1g + 0.5 * gh
    tx = (gxc - axc) / aw
    ty = (gyc - ayc) / ah
    tw = jnp.log(gw / aw)
    th = jnp.log(gh / ah)
    reg_ref[b] = jnp.concatenate([tx, ty, tw, th], axis=0)

    cls_w = (labels >= 0.0).astype(f32)
    reg_w = (labels == 1.0).astype(f32)
    clst_ref[b:b + 1, :] = cls_t
    cls_t_rows.append(cls_t)
    cls_w_rows.append(cls_w)
    reg_w_rows.append(reg_w)

  cls_t_all = jnp.concatenate(cls_t_rows, axis=0)
  cls_w_all = jnp.concatenate(cls_w_rows, axis=0)
  reg_w_all = jnp.concatenate(reg_w_rows, axis=0)
  score = score_ref[...]

  fg_elig = ((cls_t_all > 0.0) & (cls_w_all > 0.0)).astype(f32)
  bg_elig = ((cls_t_all == 0.0) & (cls_w_all > 0.0)).astype(f32)
  elig = jnp.concatenate([fg_elig, bg_elig], axis=0) > 0.0
  s_f = jnp.where(elig, jnp.concatenate([score, score], axis=0),
                  -jnp.inf)
  bits = lax.bitcast_convert_type(s_f, jnp.int32)
  skey_ref[...] = jnp.where(bits >= 0, bits, bits ^ jnp.int32(0x7FFFFFFF))
  clsw_ref[...] = cls_w_all
  regw_ref[...] = reg_w_all


def _lane_sum(x):
  total = x[0]
  for l in range(1, 16):
    total = total + x[l]
  return total


def _count_ge(keys_v, nchunk, cand):
  def body(i, accs):
    base = i * 80
    new = []
    for j, a in enumerate(accs):
      v = keys_v[pl.ds(base + j * 16, 16)]
      new.append(a + jnp.where(v >= cand, 1, 0).astype(jnp.int32))
    return tuple(new)

  zero = jnp.zeros((16,), jnp.int32)
  accs = lax.fori_loop(0, nchunk // 5, body, (zero,) * 5)
  total = accs[0] + accs[1] + accs[2] + accs[3] + accs[4]
  return _lane_sum(total)


def _lane_prefix_excl(x):
  idx = lax.broadcasted_iota(jnp.int32, (16,), 0)
  dnums = lax.GatherDimensionNumbers(
      offset_dims=(), collapsed_slice_dims=(0,), start_index_map=(0,))
  incl = x
  for step in (1, 2, 4, 8):
    j = jnp.maximum(idx - step, 0)
    shifted = lax.gather(incl, j[:, None], dnums, (1,),
                         mode=lax.GatherScatterMode.PROMISE_IN_BOUNDS)
    incl = incl + jnp.where(idx >= step, shifted, 0)
  return incl - x


def _subsample_sc_body(skey_hbm, clsw_hbm, regw_hbm, clsw_out, regw_out,
                       keys_v, bkey_v, bpos_v, mask_v, tmp_v, wts_v, shared, *,
                       num_fg, num_bg, K):
  cid = lax.axis_index("c")
  sid = lax.axis_index("s")
  nchunk = K // 16
  active = (cid == 0) & (sid < 8)
  IM = jnp.int32(_INT_MIN)

  def bit_const(p):
    return jnp.int32(_INT_MIN) if p == 31 else jnp.int32(1 << p)

  def count_ge_band(nchunks, cand_s):
    def body(i, acc):
      for u in range(5):
        v = bkey_v[pl.ds((i * 5 + u) * 16, 16)]
        acc = acc + jnp.where(v >= cand_s, 1, 0).astype(jnp.int32)
      return acc

    acc = lax.fori_loop(0, (nchunks + 4) // 5, body,
                        jnp.zeros((16,), jnp.int32))
    return _lane_sum(acc)

  def compact(nchunks, shift, prefix, from_full):
    pshift = lax.shift_right_logical(prefix, shift)
    iota16 = lax.broadcasted_iota(jnp.int32, (16,), 0)

    def one(i, carry):
      off, bminv, bmaxv = carry
      if from_full:
        v = keys_v[pl.ds(i * 16, 16)]
        p = iota16 + i * 16
      else:
        v = bkey_v[pl.ds(i * 16, 16)]
        p = bpos_v[pl.ds(i * 16, 16)]
      u = v ^ IM
      match = (lax.shift_right_logical(u, shift) == pshift) & (v != IM)
      plsc.store_compressed(bkey_v.at[pl.ds(off, 16)], v, mask=match)
      plsc.store_compressed(bpos_v.at[pl.ds(off, 16)], p, mask=match)
      npop = plsc.all_reduce_population_count(match)[0]
      bminv = jnp.minimum(bminv, jnp.where(match, v, 2147483647))
      bmaxv = jnp.maximum(bmaxv, jnp.where(match, v, _INT_MIN))
      return off + npop, bminv, bmaxv

    def body(i, carry):
      for u in range(5):
        carry = one(i * 5 + u, carry)
      return carry

    off, bminv, bmaxv = lax.fori_loop(
        0, (nchunks + 4) // 5,
        body,
        (jnp.int32(0), jnp.full((16,), 2147483647, jnp.int32),
         jnp.full((16,), _INT_MIN, jnp.int32)))
    for u in range(5):
      bkey_v[pl.ds(off + u * 16, 16)] = jnp.full((16,), _INT_MIN, jnp.int32)
    bmin = bminv[0]
    bmax = bmaxv[0]
    for l in range(1, 16):
      bmin = jnp.minimum(bmin, bminv[l])
      bmax = jnp.maximum(bmax, bmaxv[l])
    return off, bmin, bmax

  @pl.when(active)
  def _phase1():
    kk = jnp.where(sid < 4, num_fg, num_bg)
    pltpu.sync_copy(skey_hbm.at[sid], keys_v)

    prefix = jnp.int32(0)
    krem = kk
    n_above = jnp.int32(0)
    nband = jnp.int32(K)
    for shift in (28, 24, 20, 16, 12, 8, 4, 0):
      for p in range(shift + 3, shift - 1, -1):
        cand = prefix | bit_const(p)
        cand_s = cand ^ IM
        if shift == 28:
          cnt = _count_ge(keys_v, nchunk, cand_s)
        else:
          cnt = count_ge_band((nband + 15) >> 4, cand_s)
        band_cnt = cnt - n_above
        take = band_cnt >= krem
        prefix = jnp.where(take, cand, prefix)
        krem = jnp.where(take, krem, krem - band_cnt)
        n_above = jnp.where(take, n_above, cnt)
      if shift > 0:
        nin = nchunk if shift == 28 else (nband + 15) >> 4
        nband, bmin, bmax = compact(nin, shift, prefix, shift == 28)
        n_above = jnp.int32(0)
        collapse = (nband > 0) & (bmin == bmax)
        prefix = jnp.where(collapse, bmin ^ IM, prefix)
        nband = jnp.where(collapse, 0, nband)

    thr = prefix ^ IM
    need = krem

    def mask_one(i, tcnt):
      v = keys_v[pl.ds(i * 16, 16)]
      tie = v == thr
      tie_i = tie.astype(jnp.int32)
      excl = _lane_prefix_excl(tie_i)
      sel = (v > thr) | (tie & ((tcnt + excl) < need))
      sel = sel & (v != _NEG_INF_KEY)
      mask_v[pl.ds(i * 16, 16)] = sel.astype(jnp.float32)
      return tcnt + excl[15] + tie_i[15]

    def mask_body(i, tcnt):
      for u in range(5):
        tcnt = mask_one(i * 5 + u, tcnt)
      return tcnt

    lax.fori_loop(0, nchunk // 5, mask_body, jnp.int32(0))
    pltpu.sync_copy(mask_v, shared.at[sid])

  plsc.subcore_barrier()

  @pl.when(active & (sid < 4))
  def _phase2():
    pltpu.sync_copy(shared.at[sid + 4], tmp_v)
    pltpu.sync_copy(clsw_hbm.at[sid], wts_v)

    def comb_body(i, _):
      for u in range(5):
        d = pl.ds((i * 5 + u) * 16, 16)
        m = jnp.maximum(mask_v[d], tmp_v[d])
        mask_v[d] = m
        wts_v[d] = wts_v[d] * m
      return 0

    lax.fori_loop(0, nchunk // 5, comb_body, 0)
    pltpu.sync_copy(wts_v, clsw_out.at[sid])
    pltpu.sync_copy(regw_hbm.at[sid], wts_v)

    def mul_body(i, _):
      for u in range(5):
        d = pl.ds((i * 5 + u) * 16, 16)
        wts_v[d] = wts_v[d] * mask_v[d]
      return 0

    lax.fori_loop(0, nchunk // 5, mul_body, 0)
    pltpu.sync_copy(wts_v, regw_out.at[sid])


def _subsample_sc(skey, cls_w, reg_w):
  R, K = skey.shape
  B = R // 2
  mesh = plsc.VectorSubcoreMesh(core_axis_name="c", subcore_axis_name="s")
  body = functools.partial(_subsample_sc_body, num_fg=_NUM_FG, num_bg=_NUM_BG,
                           K=K)
  f = pl.kernel(
      body,
      mesh=mesh,
      compiler_params=pltpu.CompilerParams(needs_layout_passes=False),
      out_type=(
          jax.ShapeDtypeStruct((B, K), jnp.float32),
          jax.ShapeDtypeStruct((B, K), jnp.float32),
      ),
      scratch_types=[
          pltpu.VMEM((K,), jnp.int32),
          pltpu.VMEM((K + 80,), jnp.int32),
          pltpu.VMEM((K + 80,), jnp.int32),
          pltpu.VMEM((K,), jnp.float32),
          pltpu.VMEM((K,), jnp.float32),
          pltpu.VMEM((K,), jnp.float32),
          pltpu.VMEM_SHARED((8, K), jnp.float32),
      ],
  )
  return f(skey, cls_w, reg_w)


@jax.jit
def kernel(anchors, rpn_cls_score, gt_boxes, gt_labels):
  K = anchors.shape[0]
  B, M, _ = gt_boxes.shape
  anchors_t = anchors.T
  gl_f = gt_labels.astype(jnp.float32)[..., None]
  cls_t, reg, cls_w_raw, reg_w_raw, skey = pl.pallas_call(
      _assign_kernel,
      out_shape=(
          jax.ShapeDtypeStruct((B, K), jnp.float32),
          jax.ShapeDtypeStruct((B, 4, K), jnp.float32),
          jax.ShapeDtypeStruct((B, K), jnp.float32),
          jax.ShapeDtypeStruct((B, K), jnp.float32),
          jax.ShapeDtypeStruct((2 * B, K), jnp.int32),
      ),
  )(anchors_t, rpn_cls_score, gt_boxes, gl_f)
  cls_w, reg_w = _subsample_sc(skey, cls_w_raw, reg_w_raw)
  reg_t = jnp.transpose(reg, (0, 2, 1))
  return cls_t, reg_t, cls_w, reg_w
